# baseline (device time: 866167 ns/iter reference)
import jax
import jax.numpy as jnp
from jax import lax
from jax.experimental import pallas as pl
from jax.experimental.pallas import tpu as pltpu

Z = 4


def _gather_dest(dest2):
    dr, dl = dest2.shape

    def body(d_ref, dall_ref, send_sems, recv_sems):
        mx = lax.axis_index("x")
        my = lax.axis_index("y")
        mz = lax.axis_index("z")
        right = lax.rem(mz + 1, Z)
        left = lax.rem(mz + Z - 1, Z)

        barrier = pltpu.get_barrier_semaphore()
        for nbr in (left, right):
            pl.semaphore_signal(
                barrier, inc=1,
                device_id=(mx, my, nbr),
                device_id_type=pl.DeviceIdType.MESH,
            )
        pl.semaphore_wait(barrier, 2)

        dall_ref[pl.ds(mz * dr, dr), :] = d_ref[...]
        for h in range(Z - 1):
            origin = lax.rem(mz - h + Z, Z)
            rd = pltpu.make_async_remote_copy(
                src_ref=dall_ref.at[pl.ds(origin * dr, dr), :],
                dst_ref=dall_ref.at[pl.ds(origin * dr, dr), :],
                send_sem=send_sems.at[h],
                recv_sem=recv_sems.at[h],
                device_id=(mx, my, right),
                device_id_type=pl.DeviceIdType.MESH,
            )
            rd.start()
            rd.wait()

    return pl.pallas_call(
        body,
        out_shape=jax.ShapeDtypeStruct((Z * dr, dl), jnp.int32),
        in_specs=[pl.BlockSpec(memory_space=pltpu.VMEM)],
        out_specs=pl.BlockSpec(memory_space=pltpu.VMEM),
        scratch_shapes=[
            pltpu.SemaphoreType.DMA((Z - 1,)),
            pltpu.SemaphoreType.DMA((Z - 1,)),
        ],
        compiler_params=pltpu.CompilerParams(collective_id=0),
    )(dest2)


def _a2a_gather(x, idx, off):
    t, d = x.shape
    hh = t // 2

    def body(x_ref, idx_ref, off_ref, out_ref, xall_ref,
             r_send, r_recv, l_send, l_recv):
        mx = lax.axis_index("x")
        my = lax.axis_index("y")
        mz = lax.axis_index("z")
        right = lax.rem(mz + 1, Z)
        left = lax.rem(mz + Z - 1, Z)

        barrier = pltpu.get_barrier_semaphore()
        for nbr in (left, right):
            pl.semaphore_signal(
                barrier, inc=1,
                device_id=(mx, my, nbr),
                device_id_type=pl.DeviceIdType.MESH,
            )
        pl.semaphore_wait(barrier, 2)

        xall_ref[pl.ds(mz * t, t), :] = x_ref[...]

        def gather_block(k):
            def one(j, carry):
                out_ref[pl.ds(j, 1), :] = xall_ref[pl.ds(idx_ref[j], 1), :]
                return carry
            lax.fori_loop(off_ref[k], off_ref[k + 1], one, 0)

        def start_hop(h):
            o_r = lax.rem(mz - h + Z, Z)
            o_l = lax.rem(mz + h, Z)
            r = pltpu.make_async_remote_copy(
                src_ref=xall_ref.at[pl.ds(o_r * t, hh), :],
                dst_ref=xall_ref.at[pl.ds(o_r * t, hh), :],
                send_sem=r_send.at[h],
                recv_sem=r_recv.at[h],
                device_id=(mx, my, right),
                device_id_type=pl.DeviceIdType.MESH,
            )
            l = pltpu.make_async_remote_copy(
                src_ref=xall_ref.at[pl.ds(o_l * t + hh, hh), :],
                dst_ref=xall_ref.at[pl.ds(o_l * t + hh, hh), :],
                send_sem=l_send.at[h],
                recv_sem=l_recv.at[h],
                device_id=(mx, my, left),
                device_id_type=pl.DeviceIdType.MESH,
            )
            r.start()
            l.start()
            return r, l

        inflight = start_hop(0)
        gather_block(2 * mz)
        gather_block(2 * mz + 1)
        for h in range(Z - 1):
            inflight[0].wait()
            inflight[1].wait()
            if h < Z - 2:
                inflight = start_hop(h + 1)
            gather_block(2 * lax.rem(mz - 1 - h + 2 * Z, Z))
            gather_block(2 * lax.rem(mz + 1 + h, Z) + 1)

    return pl.pallas_call(
        body,
        out_shape=jax.ShapeDtypeStruct((t, d), jnp.float32),
        in_specs=[
            pl.BlockSpec(memory_space=pltpu.VMEM),
            pl.BlockSpec(memory_space=pltpu.SMEM),
            pl.BlockSpec(memory_space=pltpu.SMEM),
        ],
        out_specs=pl.BlockSpec(memory_space=pltpu.VMEM),
        scratch_shapes=[
            pltpu.VMEM((Z * t, d), jnp.float32),
            pltpu.SemaphoreType.DMA((Z - 1,)),
            pltpu.SemaphoreType.DMA((Z - 1,)),
            pltpu.SemaphoreType.DMA((Z - 1,)),
            pltpu.SemaphoreType.DMA((Z - 1,)),
        ],
        compiler_params=pltpu.CompilerParams(collective_id=1),
    )(x, idx, off)


def kernel(x, dest):
    t, d = x.shape
    dl = 128
    dr = t // dl
    dall = _gather_dest(dest.reshape(dr, dl))

    mz = lax.axis_index("z")
    dflat = dall.reshape(Z * t)
    c = jnp.cumsum((dflat == mz).astype(jnp.int32))
    idx = jnp.searchsorted(
        c, jnp.arange(1, t + 1, dtype=jnp.int32)
    ).astype(jnp.int32)
    off = jnp.searchsorted(
        idx, jnp.arange(0, Z * t + 1, t // 2, dtype=jnp.int32)
    ).astype(jnp.int32)

    return _a2a_gather(x, idx, off)


# device time: 335892 ns/iter; 2.5787x vs baseline; 2.5787x over previous
import jax
import jax.numpy as jnp
from jax import lax
from jax.experimental import pallas as pl
from jax.experimental.pallas import tpu as pltpu

Z = 4
DL = 128


def kernel(x, dest):
    t, d = x.shape
    hh = t // 2
    dr = t // DL
    dest2 = dest.reshape(dr, DL)

    def body(x_ref, d_ref, out_ref, xall_ref, dall_ref, dall_s, idx_s, off_s,
             d_send, d_recv, r_send, r_recv, l_send, l_recv, cp_sem):
        mx = lax.axis_index("x")
        my = lax.axis_index("y")
        mz = lax.axis_index("z")
        right = lax.rem(mz + 1, Z)
        left = lax.rem(mz + Z - 1, Z)

        barrier = pltpu.get_barrier_semaphore()
        for nbr in (left, right):
            pl.semaphore_signal(
                barrier, inc=1,
                device_id=(mx, my, nbr),
                device_id_type=pl.DeviceIdType.MESH,
            )
        pl.semaphore_wait(barrier, 2)

        dall_ref[pl.ds(mz * dr, dr), :] = d_ref[...]
        for h in range(Z - 1):
            origin = lax.rem(mz - h + Z, Z)
            rd = pltpu.make_async_remote_copy(
                src_ref=dall_ref.at[pl.ds(origin * dr, dr), :],
                dst_ref=dall_ref.at[pl.ds(origin * dr, dr), :],
                send_sem=d_send.at[h],
                recv_sem=d_recv.at[h],
                device_id=(mx, my, right),
                device_id_type=pl.DeviceIdType.MESH,
            )
            rd.start()
            rd.wait()
        cp = pltpu.make_async_copy(dall_ref, dall_s, cp_sem)
        cp.start()

        xall_ref[pl.ds(mz * t, t), :] = x_ref[...]

        def start_hop(h):
            o_r = lax.rem(mz - h + Z, Z)
            o_l = lax.rem(mz + h, Z)
            r = pltpu.make_async_remote_copy(
                src_ref=xall_ref.at[pl.ds(o_r * t, hh), :],
                dst_ref=xall_ref.at[pl.ds(o_r * t, hh), :],
                send_sem=r_send.at[h],
                recv_sem=r_recv.at[h],
                device_id=(mx, my, right),
                device_id_type=pl.DeviceIdType.MESH,
            )
            l = pltpu.make_async_remote_copy(
                src_ref=xall_ref.at[pl.ds(o_l * t + hh, hh), :],
                dst_ref=xall_ref.at[pl.ds(o_l * t + hh, hh), :],
                send_sem=l_send.at[h],
                recv_sem=l_recv.at[h],
                device_id=(mx, my, left),
                device_id_type=pl.DeviceIdType.MESH,
            )
            r.start()
            l.start()
            return r, l

        inflight = start_hop(0)

        cp.wait()

        def scan_i(i, cnt):
            r_i = lax.div(i, DL)
            l_i = lax.rem(i, DL)
            hit = dall_s[r_i, l_i] == mz

            @pl.when(lax.rem(i, hh) == 0)
            def _():
                off_s[lax.div(i, hh)] = cnt

            @pl.when(hit)
            def _():
                idx_s[cnt] = i

            return cnt + hit.astype(jnp.int32)

        lax.fori_loop(0, Z * t, scan_i, jnp.int32(0))
        off_s[2 * Z] = t

        def gather_block(k):
            def one(j, carry):
                out_ref[pl.ds(j, 1), :] = xall_ref[pl.ds(idx_s[j], 1), :]
                return carry
            lax.fori_loop(off_s[k], off_s[k + 1], one, 0)

        gather_block(2 * mz)
        gather_block(2 * mz + 1)
        for h in range(Z - 1):
            inflight[0].wait()
            inflight[1].wait()
            if h < Z - 2:
                inflight = start_hop(h + 1)
            gather_block(2 * lax.rem(mz - 1 - h + 2 * Z, Z))
            gather_block(2 * lax.rem(mz + 1 + h, Z) + 1)

    return pl.pallas_call(
        body,
        out_shape=jax.ShapeDtypeStruct((t, d), jnp.float32),
        in_specs=[
            pl.BlockSpec(memory_space=pltpu.VMEM),
            pl.BlockSpec(memory_space=pltpu.VMEM),
        ],
        out_specs=pl.BlockSpec(memory_space=pltpu.VMEM),
        scratch_shapes=[
            pltpu.VMEM((Z * t, d), jnp.float32),
            pltpu.VMEM((Z * dr, DL), jnp.int32),
            pltpu.SMEM((Z * dr, DL), jnp.int32),
            pltpu.SMEM((t,), jnp.int32),
            pltpu.SMEM((2 * Z + 1,), jnp.int32),
            pltpu.SemaphoreType.DMA((Z - 1,)),
            pltpu.SemaphoreType.DMA((Z - 1,)),
            pltpu.SemaphoreType.DMA((Z - 1,)),
            pltpu.SemaphoreType.DMA((Z - 1,)),
            pltpu.SemaphoreType.DMA((Z - 1,)),
            pltpu.SemaphoreType.DMA((Z - 1,)),
            pltpu.SemaphoreType.DMA,
        ],
        compiler_params=pltpu.CompilerParams(
            collective_id=0,
            vmem_limit_bytes=100 * 1024 * 1024,
        ),
    )(x, dest2)


# device time: 119698 ns/iter; 7.2363x vs baseline; 2.8062x over previous
import jax
import jax.numpy as jnp
from jax import lax
from jax.experimental import pallas as pl
from jax.experimental.pallas import tpu as pltpu

Z = 4
DL = 128
MAXQ = 16


def kernel(x, dest):
    t, d = x.shape
    dr = t // DL
    dest2 = dest.reshape(dr, DL)

    def body(x_ref, d_ref, out_ref, cmat_ref, cnt_s, dest_s, runn_s,
             c_send, c_recv, row_send, in_sem, own_sem, cp_sems):
        mx = lax.axis_index("x")
        my = lax.axis_index("y")
        mz = lax.axis_index("z")

        def row_rdma(src_i, dst_j, dev_z):
            return pltpu.make_async_remote_copy(
                src_ref=x_ref.at[pl.ds(src_i, 1), :],
                dst_ref=out_ref.at[pl.ds(dst_j, 1), :],
                send_sem=row_send,
                recv_sem=in_sem,
                device_id=(mx, my, dev_z),
                device_id_type=pl.DeviceIdType.MESH,
            )

        def own_copy(src_i, dst_j):
            return pltpu.make_async_copy(
                x_ref.at[pl.ds(src_i, 1), :],
                out_ref.at[pl.ds(dst_j, 1), :],
                own_sem,
            )

        barrier = pltpu.get_barrier_semaphore()
        for j in range(1, Z):
            pl.semaphore_signal(
                barrier, inc=1,
                device_id=(mx, my, lax.rem(mz + j, Z)),
                device_id_type=pl.DeviceIdType.MESH,
            )
        pl.semaphore_wait(barrier, Z - 1)

        cp_d = pltpu.make_async_copy(d_ref, dest_s, cp_sems.at[0])
        cp_d.start()

        lane = lax.broadcasted_iota(jnp.int32, (1, DL), 1)
        row = jnp.zeros((1, DL), jnp.int32)
        for r in range(Z):
            c_r = jnp.sum((d_ref[...] == r).astype(jnp.int32))
            row = row + jnp.where(lane == r, c_r, 0)
        cmat_ref[pl.ds(mz, 1), :] = row

        def cnt_rdma(dev_z):
            return pltpu.make_async_remote_copy(
                src_ref=cmat_ref.at[pl.ds(mz, 1), :],
                dst_ref=cmat_ref.at[pl.ds(mz, 1), :],
                send_sem=c_send,
                recv_sem=c_recv,
                device_id=(mx, my, dev_z),
                device_id_type=pl.DeviceIdType.MESH,
            )

        for j in range(1, Z):
            cnt_rdma(lax.rem(mz + j, Z)).start()
        for j in range(Z - 1):
            cnt_rdma(mz).wait_recv()
        for j in range(Z - 1):
            cnt_rdma(mz).wait_send()
        cp_c = pltpu.make_async_copy(cmat_ref, cnt_s, cp_sems.at[1])
        cp_c.start()
        cp_c.wait()
        cp_d.wait()

        for r in range(Z):
            base_r = jnp.int32(0)
            for s in range(Z):
                base_r = base_r + jnp.where(s < mz, cnt_s[s, r], 0)
            runn_s[r] = base_r

        def one(l, carry, rr):
            n_rem, w_rem, n_loc, w_loc = carry
            i = rr * DL + l
            dloc = dest_s[rr, l]
            pos = runn_s[dloc]
            runn_s[dloc] = pos + 1
            is_rem = (dloc != mz).astype(jnp.int32)

            @pl.when(dloc != mz)
            def _():
                row_rdma(i, pos, dloc).start()

            @pl.when(dloc == mz)
            def _():
                own_copy(i, pos).start()

            n_rem = n_rem + is_rem
            n_loc = n_loc + 1 - is_rem
            wr = (n_rem - w_rem >= MAXQ).astype(jnp.int32)
            wl = (n_loc - w_loc >= MAXQ).astype(jnp.int32)

            @pl.when(wr == 1)
            def _():
                row_rdma(0, 0, mz).wait_send()

            @pl.when(wl == 1)
            def _():
                own_copy(0, 0).wait()

            return n_rem, w_rem + wr, n_loc, w_loc + wl

        carry = (jnp.int32(0),) * 4
        for rr in range(dr):
            carry = lax.fori_loop(
                0, DL, lambda l, c, _rr=rr: one(l, c, _rr), carry
            )
        n_rem, w_rem, n_loc, w_loc = carry

        def drain(n, wait_fn):
            def step(_, c):
                wait_fn()
                return c
            lax.fori_loop(0, n, step, 0)

        drain(n_rem - w_rem, lambda: row_rdma(0, 0, mz).wait_send())
        drain(n_loc - w_loc, lambda: own_copy(0, 0).wait())

        n_in = jnp.int32(0)
        for s in range(Z):
            n_in = n_in + jnp.where(s == mz, 0, cnt_s[s, mz])
        drain(n_in, lambda: row_rdma(0, 0, mz).wait_recv())

    return pl.pallas_call(
        body,
        out_shape=jax.ShapeDtypeStruct((t, d), jnp.float32),
        in_specs=[
            pl.BlockSpec(memory_space=pltpu.VMEM),
            pl.BlockSpec(memory_space=pltpu.VMEM),
        ],
        out_specs=pl.BlockSpec(memory_space=pltpu.VMEM),
        scratch_shapes=[
            pltpu.VMEM((Z, DL), jnp.int32),
            pltpu.SMEM((Z, DL), jnp.int32),
            pltpu.SMEM((dr, DL), jnp.int32),
            pltpu.SMEM((Z,), jnp.int32),
            pltpu.SemaphoreType.DMA,
            pltpu.SemaphoreType.DMA,
            pltpu.SemaphoreType.DMA,
            pltpu.SemaphoreType.DMA,
            pltpu.SemaphoreType.DMA,
            pltpu.SemaphoreType.DMA((2,)),
        ],
        compiler_params=pltpu.CompilerParams(
            collective_id=0,
            vmem_limit_bytes=100 * 1024 * 1024,
        ),
    )(x, dest2)
